# baseline (speedup 1.0000x reference)
import jax
import jax.numpy as jnp
from jax.experimental import pallas as pl
from jax.experimental.pallas import tpu as pltpu

BATCH = 4
SEQ = 8192
RANK = 4096
NUM_EXPERTS = 64
TOP_K = 8

M_BLK = 1024


SUB = 128


def _topk_sub(probs):
    iota_f = jax.lax.broadcasted_iota(jnp.int32, probs.shape, 1).astype(
        jnp.float32)
    work = probs
    ws = []
    idxs = []
    for _ in range(TOP_K):
        mj = jnp.max(work, axis=-1, keepdims=True)
        ij = jnp.min(jnp.where(work == mj, iota_f, float(NUM_EXPERTS)),
                     axis=-1, keepdims=True)
        ws.append(mj)
        idxs.append(ij)
        work = jnp.where(iota_f == ij, -1.0, work)
    return (jnp.concatenate(ws, axis=1),
            jnp.concatenate(idxs, axis=1).astype(jnp.int32))


def _router_body(z_ref, wt_ref, probs_ref, tw_ref, ti_ref):
    logits = jnp.dot(z_ref[...], wt_ref[...],
                     preferred_element_type=jnp.float32)
    m = jnp.max(logits, axis=-1, keepdims=True)
    e = jnp.exp(logits - m)
    s = jnp.sum(e, axis=-1, keepdims=True)
    probs = e / s
    probs_ref[...] = probs

    for r in range(M_BLK // SUB):
        sl = pl.ds(r * SUB, SUB)
        tw, ti = _topk_sub(probs[r * SUB:(r + 1) * SUB, :])
        tw_ref[sl, :] = tw
        ti_ref[sl, :] = ti


def kernel(z, W):
    tokens = BATCH * SEQ
    zr = z.reshape(tokens, RANK)
    wt = W.T

    grid = (tokens // M_BLK,)
    probs, tw, ti = pl.pallas_call(
        _router_body,
        grid=grid,
        in_specs=[
            pl.BlockSpec((M_BLK, RANK), lambda i: (i, 0)),
            pl.BlockSpec((RANK, NUM_EXPERTS), lambda i: (0, 0)),
        ],
        out_specs=[
            pl.BlockSpec((M_BLK, NUM_EXPERTS), lambda i: (i, 0)),
            pl.BlockSpec((M_BLK, TOP_K), lambda i: (i, 0)),
            pl.BlockSpec((M_BLK, TOP_K), lambda i: (i, 0)),
        ],
        out_shape=[
            jax.ShapeDtypeStruct((tokens, NUM_EXPERTS), jnp.float32),
            jax.ShapeDtypeStruct((tokens, TOP_K), jnp.float32),
            jax.ShapeDtypeStruct((tokens, TOP_K), jnp.int32),
        ],
        compiler_params=pltpu.CompilerParams(
            dimension_semantics=("parallel",),
        ),
    )(zr, wt)

    return (tw.reshape(BATCH, SEQ, TOP_K),
            ti.reshape(BATCH, SEQ, TOP_K),
            probs.reshape(BATCH, SEQ, NUM_EXPERTS))


# speedup vs baseline: 1.0022x; 1.0022x over previous
import jax
import jax.numpy as jnp
from jax.experimental import pallas as pl
from jax.experimental.pallas import tpu as pltpu

BATCH = 4
SEQ = 8192
RANK = 4096
NUM_EXPERTS = 64
TOP_K = 8

M_BLK = 1024


SUB = 256


def _topk_sub(probs):
    iota_f = jax.lax.broadcasted_iota(jnp.int32, probs.shape, 1).astype(
        jnp.float32)
    work = probs
    ws = []
    idxs = []
    for t in range(TOP_K):
        mj = jnp.max(work, axis=-1, keepdims=True)
        ij = jnp.min(jnp.where(work == mj, iota_f, float(NUM_EXPERTS)),
                     axis=-1, keepdims=True)
        ws.append(mj)
        idxs.append(ij)
        if t + 1 < TOP_K:
            work = jnp.where(iota_f == ij, -1.0, work)
    return (jnp.concatenate(ws, axis=1),
            jnp.concatenate(idxs, axis=1).astype(jnp.int32))


def _router_body(z_ref, wt_ref, probs_ref, tw_ref, ti_ref):
    logits = jnp.dot(z_ref[...], wt_ref[...],
                     preferred_element_type=jnp.float32)
    m = jnp.max(logits, axis=-1, keepdims=True)
    e = jnp.exp(logits - m)
    s = jnp.sum(e, axis=-1, keepdims=True)
    probs = e / s
    probs_ref[...] = probs

    for r in range(M_BLK // SUB):
        sl = pl.ds(r * SUB, SUB)
        tw, ti = _topk_sub(probs[r * SUB:(r + 1) * SUB, :])
        tw_ref[sl, :] = tw
        ti_ref[sl, :] = ti


def kernel(z, W):
    tokens = BATCH * SEQ
    zr = z.reshape(tokens, RANK)
    wt = W.T

    grid = (tokens // M_BLK,)
    probs, tw, ti = pl.pallas_call(
        _router_body,
        grid=grid,
        in_specs=[
            pl.BlockSpec((M_BLK, RANK), lambda i: (i, 0)),
            pl.BlockSpec((RANK, NUM_EXPERTS), lambda i: (0, 0)),
        ],
        out_specs=[
            pl.BlockSpec((M_BLK, NUM_EXPERTS), lambda i: (i, 0)),
            pl.BlockSpec((M_BLK, TOP_K), lambda i: (i, 0)),
            pl.BlockSpec((M_BLK, TOP_K), lambda i: (i, 0)),
        ],
        out_shape=[
            jax.ShapeDtypeStruct((tokens, NUM_EXPERTS), jnp.float32),
            jax.ShapeDtypeStruct((tokens, TOP_K), jnp.float32),
            jax.ShapeDtypeStruct((tokens, TOP_K), jnp.int32),
        ],
        compiler_params=pltpu.CompilerParams(
            dimension_semantics=("parallel",),
        ),
    )(zr, wt)

    return (tw.reshape(BATCH, SEQ, TOP_K),
            ti.reshape(BATCH, SEQ, TOP_K),
            probs.reshape(BATCH, SEQ, NUM_EXPERTS))
